# baseline (speedup 1.0000x reference)
import jax
import jax.numpy as jnp
from jax.experimental import pallas as pl
from jax.experimental.pallas import tpu as pltpu
from jax.experimental.pallas import tpu_sc as plsc

T = 64
D = 1024
E = 64
DFF = 1024
LANES = 16
NCHUNK = E // LANES
TOK_PER_SUB = T // 16


def _lane_idx(c):
    return jnp.arange(LANES, dtype=jnp.int32) + jnp.full((LANES,), LANES * c, jnp.int32)


def _sc_routing(gate_hbm, w_hbm, sched_hbm, g_v, w_v, sched_v, ord_v):
    cid = jax.lax.axis_index("c")
    sid = jax.lax.axis_index("s")
    NEG = jnp.float32(-3.0e38)
    ones = jnp.full((LANES,), 1, jnp.int32)
    zeros = jnp.full((LANES,), 0, jnp.int32)

    @pl.when((cid == 0) & (sid == 0))
    def _route_tokens():
        pltpu.sync_copy(gate_hbm, g_v)

        def _token_body(j, accs):
            chunks = [g_v[j, pl.ds(LANES * c, LANES)] for c in range(NCHUNK)]
            m1 = jnp.maximum(jnp.maximum(jnp.max(chunks[0]), jnp.max(chunks[1])),
                             jnp.maximum(jnp.max(chunks[2]), jnp.max(chunks[3])))
            cands = [jnp.where(chunks[c] == m1, _lane_idx(c), jnp.full((LANES,), E, jnp.int32))
                     for c in range(NCHUNK)]
            a1 = jnp.minimum(jnp.minimum(jnp.min(cands[0]), jnp.min(cands[1])),
                             jnp.minimum(jnp.min(cands[2]), jnp.min(cands[3])))
            rest = [jnp.where(_lane_idx(c) == a1, jnp.full((LANES,), NEG), chunks[c])
                    for c in range(NCHUNK)]
            m2 = jnp.maximum(jnp.maximum(jnp.max(rest[0]), jnp.max(rest[1])),
                             jnp.maximum(jnp.max(rest[2]), jnp.max(rest[3])))
            cands2 = [jnp.where(rest[c] == m2, _lane_idx(c), jnp.full((LANES,), E, jnp.int32))
                      for c in range(NCHUNK)]
            a2 = jnp.minimum(jnp.minimum(jnp.min(cands2[0]), jnp.min(cands2[1])),
                             jnp.minimum(jnp.min(cands2[2]), jnp.min(cands2[3])))
            p2 = jnp.exp(jnp.full((LANES,), m2 - m1))
            w1 = 1.0 / (1.0 + p2)
            w2 = p2 / (1.0 + p2)
            new_accs = []
            for c in range(NCHUNK):
                sel1 = _lane_idx(c) == a1
                sel2 = _lane_idx(c) == a2
                wc = (jnp.where(sel1, w1, jnp.full((LANES,), 0.0, jnp.float32))
                      + jnp.where(sel2, w2, jnp.full((LANES,), 0.0, jnp.float32)))
                w_v[j, pl.ds(LANES * c, LANES)] = wc
                new_accs.append(accs[c] | jnp.where(sel1 | sel2, ones, zeros))
            return tuple(new_accs)

        accs = jax.lax.fori_loop(0, T, _token_body, (zeros, zeros, zeros, zeros))
        pltpu.sync_copy(w_v, w_hbm)
        lasts = [jnp.max(jnp.where(accs[c] == 1, _lane_idx(c),
                                   jnp.full((LANES,), -1, jnp.int32)))
                 for c in range(NCHUNK)]
        last = jnp.maximum(jnp.maximum(lasts[0], lasts[1]),
                           jnp.maximum(lasts[2], lasts[3]))
        for c in range(NCHUNK):
            ord_v[pl.ds(LANES * c, LANES)] = jnp.full((LANES,), last)
        off = jnp.int32(0)
        n = jnp.int32(0)
        for c in range(NCHUNK):
            mask = accs[c] == 1
            pos = plsc.cumsum(accs[c]) - 1 + off
            pos = jnp.where(mask, pos, jnp.full((LANES,), 0, jnp.int32))
            plsc.store_scatter(ord_v, [pos], _lane_idx(c), mask=mask)
            cnt = jnp.sum(accs[c])
            off = off + cnt
            n = n + cnt
        for c in range(NCHUNK):
            sched_v[0, pl.ds(LANES * c, LANES)] = ord_v[pl.ds(LANES * c, LANES)]
            sched_v[1, pl.ds(LANES * c, LANES)] = jnp.where(_lane_idx(c) < n, ones, zeros)
            for r in range(2, 8):
                sched_v[r, pl.ds(LANES * c, LANES)] = zeros
        pltpu.sync_copy(sched_v, sched_hbm)


def _moe_step(sched_ref, x_ref, w_ref, gp_ref, up_ref, dp_ref, out_ref):
    i = pl.program_id(0)

    @pl.when(i == 0)
    def _init():
        out_ref[...] = jnp.zeros_like(out_ref)

    @pl.when(sched_ref[1, i] == 1)
    def _compute():
        x = x_ref[...]
        gate = jax.lax.dot_general(
            x, gp_ref[0, 0], (((1,), (1,)), ((), ())),
            preferred_element_type=jnp.float32)
        up = jax.lax.dot_general(
            x, up_ref[0, 0], (((1,), (1,)), ((), ())),
            preferred_element_type=jnp.float32)
        h = gate * jax.nn.sigmoid(gate) * up
        oe = jax.lax.dot_general(
            h, dp_ref[0], (((1,), (1,)), ((), ())),
            preferred_element_type=jnp.float32)
        e_id = sched_ref[0, i]
        eidx = jax.lax.broadcasted_iota(jnp.int32, (T, E), 1)
        we = jnp.sum(jnp.where(eidx == e_id, w_ref[...], 0.0), axis=1, keepdims=True)
        out_ref[...] += we * oe


@jax.jit
def kernel(x, gating_output, gate_up_proj, down_proj):
    w_te, sched = pl.kernel(
        _sc_routing,
        out_type=[
            jax.ShapeDtypeStruct((T, E), jnp.float32),
            jax.ShapeDtypeStruct((8, E), jnp.int32),
        ],
        mesh=plsc.VectorSubcoreMesh(core_axis_name="c", subcore_axis_name="s"),
        scratch_types=[
            pltpu.VMEM((T, E), jnp.float32),
            pltpu.VMEM((T, E), jnp.float32),
            pltpu.VMEM((8, E), jnp.int32),
            pltpu.VMEM((E,), jnp.int32),
        ],
        compiler_params=pltpu.CompilerParams(needs_layout_passes=False),
    )(gating_output)

    gup4 = gate_up_proj.reshape(E, 2, DFF, D)
    return pl.pallas_call(
        _moe_step,
        grid_spec=pltpu.PrefetchScalarGridSpec(
            num_scalar_prefetch=1,
            grid=(E,),
            in_specs=[
                pl.BlockSpec((T, D), lambda i, s: (0, 0)),
                pl.BlockSpec((T, E), lambda i, s: (0, 0)),
                pl.BlockSpec((1, 1, DFF, D), lambda i, s: (s[0, i], 0, 0, 0)),
                pl.BlockSpec((1, 1, DFF, D), lambda i, s: (s[0, i], 1, 0, 0)),
                pl.BlockSpec((1, D, DFF), lambda i, s: (s[0, i], 0, 0)),
            ],
            out_specs=pl.BlockSpec((T, D), lambda i, s: (0, 0)),
        ),
        out_shape=jax.ShapeDtypeStruct((T, D), jnp.float32),
    )(sched, x, w_te, gup4, gup4, down_proj)
